# Initial kernel scaffold; baseline (speedup 1.0000x reference)
import jax
import jax.numpy as jnp
from jax.experimental import pallas as pl


def kernel(x, edge_index, W, b, gamma, beta):
    raise NotImplementedError("write your pallas kernel here")


# speedup vs baseline: 37.9041x; 37.9041x over previous
import functools

import jax
import jax.numpy as jnp
from jax import lax
from jax.experimental import pallas as pl
from jax.experimental.pallas import tpu as pltpu
from jax.experimental.pallas import tpu_sc as plsc

NC = 2
NS = 16
NW = NC * NS
B = 128

_mesh = plsc.VectorSubcoreMesh(
    core_axis_name="c", subcore_axis_name="s", num_cores=NC, num_subcores=NS)


def _deg_kernel(npad, nb):
  rows_per_tile = npad // NS

  @functools.partial(
      pl.kernel,
      out_type=jax.ShapeDtypeStruct((NC, npad), jnp.float32),
      mesh=_mesh,
      scratch_types=[
          pltpu.VMEM((nb, B), jnp.int32),
          pltpu.VMEM((B,), jnp.float32),
          pltpu.VMEM((rows_per_tile,), jnp.float32),
          pltpu.VMEM_SHARED((npad,), jnp.float32),
      ],
  )
  def k(dst_hbm, out_hbm, dst_loc, ones_v, stage_v, deg_sp):
    cid = lax.axis_index("c")
    sid = lax.axis_index("s")
    wid = cid * NS + sid

    @pl.loop(0, rows_per_tile // 16)
    def _(i):
      stage_v[pl.ds(i * 16, 16)] = jnp.zeros((16,), jnp.float32)

    for j in range(B // 16):
      ones_v[pl.ds(j * 16, 16)] = jnp.ones((16,), jnp.float32)

    pltpu.sync_copy(stage_v, deg_sp.at[pl.ds(sid * rows_per_tile, rows_per_tile)])
    pltpu.sync_copy(dst_hbm.at[wid], dst_loc)
    plsc.subcore_barrier()

    @pl.loop(0, nb)
    def _(b):
      pltpu.sync_copy(ones_v, deg_sp.at[dst_loc.at[b]], add=True)

    plsc.subcore_barrier()
    base = sid * rows_per_tile
    pltpu.sync_copy(deg_sp.at[pl.ds(base, rows_per_tile)], stage_v)
    pltpu.sync_copy(stage_v, out_hbm.at[cid, pl.ds(base, rows_per_tile)])

  return k


def _gs_kernel(npad, nb, d):
  rows_per_tile = npad // NS
  chunks = rows_per_tile // B
  nhalf = nb // 2

  @functools.partial(
      pl.kernel,
      out_type=jax.ShapeDtypeStruct((NC, npad, d), jnp.float32),
      mesh=_mesh,
      scratch_types=[
          pltpu.VMEM((nhalf, B), jnp.int32),
          pltpu.VMEM((nhalf, B), jnp.int32),
          pltpu.VMEM((B, d), jnp.float32),
          pltpu.VMEM((B, d), jnp.float32),
          pltpu.VMEM_SHARED((npad, d), jnp.float32),
          pltpu.SemaphoreType.DMA,
          pltpu.SemaphoreType.DMA,
      ],
  )
  def k(g_hbm, src_hbm, dst_hbm, out_hbm,
        src_loc, dst_loc, buf0, buf1, acc_sp, sem0, sem1):
    cid = lax.axis_index("c")
    sid = lax.axis_index("s")
    wid = cid * NS + sid
    base = sid * rows_per_tile

    @pl.loop(0, B)
    def _(r):
      for j in range(d // 16):
        buf0[r, pl.ds(j * 16, 16)] = jnp.zeros((16,), jnp.float32)

    @pl.loop(0, chunks)
    def _(kk):
      pltpu.sync_copy(buf0, acc_sp.at[pl.ds(base + kk * B, B)])

    plsc.subcore_barrier()

    for ph in range(2):
      pltpu.sync_copy(src_hbm.at[wid, pl.ds(ph * nhalf, nhalf)], src_loc)
      pltpu.sync_copy(dst_hbm.at[wid, pl.ds(ph * nhalf, nhalf)], dst_loc)

      pltpu.async_copy(g_hbm.at[src_loc.at[0]], buf0, sem0)

      @pl.loop(0, nhalf // 2)
      def _(i):
        b = i * 2
        pltpu.make_async_copy(g_hbm.at[src_loc.at[b]], buf0, sem0).wait()
        pltpu.async_copy(g_hbm.at[src_loc.at[b + 1]], buf1, sem1)
        pltpu.sync_copy(buf0, acc_sp.at[dst_loc.at[b]], add=True)
        pltpu.make_async_copy(g_hbm.at[src_loc.at[b + 1]], buf1, sem1).wait()
        b2 = jnp.minimum(b + 2, nhalf - 1)
        pltpu.async_copy(g_hbm.at[src_loc.at[b2]], buf0, sem0)
        pltpu.sync_copy(buf1, acc_sp.at[dst_loc.at[b + 1]], add=True)

      pltpu.make_async_copy(g_hbm.at[src_loc.at[0]], buf0, sem0).wait()

    plsc.subcore_barrier()

    @pl.loop(0, chunks)
    def _(kk):
      pltpu.sync_copy(acc_sp.at[pl.ds(base + kk * B, B)], buf0)
      pltpu.sync_copy(buf0, out_hbm.at[cid, pl.ds(base + kk * B, B)])

  return k


def _scale_body(x_ref, w_ref, degp_ref, g_ref):
  deg = degp_ref[0] + degp_ref[1] + 1.0
  dinv = lax.rsqrt(deg)
  h = jnp.dot(x_ref[...], w_ref[...], preferred_element_type=jnp.float32)
  g_ref[...] = h * dinv


def _epilogue_body(n, sp_ref, g_ref, degp_ref, b_ref, gamma_ref, beta_ref, o_ref):
  deg = degp_ref[0] + degp_ref[1] + 1.0
  dinv = lax.rsqrt(deg)
  t = (sp_ref[0] + sp_ref[1] + g_ref[...]) * (dinv / deg) + b_ref[...]
  t = jnp.where(t >= 0.0, t, 0.01 * t)
  tv = t[:n]
  mean = jnp.sum(tv, axis=0, keepdims=True) * (1.0 / n)
  dev = tv - mean
  var = jnp.sum(dev * dev, axis=0, keepdims=True) * (1.0 / n)
  o_ref[...] = dev * lax.rsqrt(var + 1e-5) * gamma_ref[...] + beta_ref[...]


def kernel(x, edge_index, W, b, gamma, beta):
  n, d_in = x.shape
  d = W.shape[1]
  e = edge_index.shape[1]

  npad = ((n + NS * B - 1) // (NS * B)) * (NS * B)
  ept = -(-e // NW)
  nb = -(-ept // B)
  nb = ((nb + 3) // 4) * 4
  tot = NW * nb * B

  src = edge_index[0].astype(jnp.int32)
  dst = edge_index[1].astype(jnp.int32)
  pad = tot - e
  pad_src = (jnp.arange(pad, dtype=jnp.int32) * 131) % n
  pad_dst = n + (jnp.arange(pad, dtype=jnp.int32) % (npad - n))
  src3 = jnp.concatenate([src, pad_src]).reshape(NW, nb, B)
  dst3 = jnp.concatenate([dst, pad_dst]).reshape(NW, nb, B)

  x_pad = jnp.pad(x, ((0, npad - n), (0, 0)))

  degp = _deg_kernel(npad, nb)(dst3)
  degp3 = degp.reshape(NC, npad, 1)

  g = pl.pallas_call(
      _scale_body,
      out_shape=jax.ShapeDtypeStruct((npad, d), jnp.float32),
  )(x_pad, W, degp3)

  sp = _gs_kernel(npad, nb, d)(g, src3, dst3)

  out = pl.pallas_call(
      functools.partial(_epilogue_body, n),
      out_shape=jax.ShapeDtypeStruct((n, d), jnp.float32),
  )(sp, g, degp3, b.reshape(1, d), gamma.reshape(1, d), beta.reshape(1, d))
  return out
